# baseline (speedup 1.0000x reference)
import functools

import jax
import jax.numpy as jnp
from jax import lax
from jax.experimental import pallas as pl
from jax.experimental.pallas import tpu as pltpu
from jax.experimental.pallas import tpu_sc as plsc

N = 10000
NPAD = 10240
E = 320000
F = 16
K = 10
NS = 16
EPT = 20480
CH = 256
GRP = 2
NSETS = 4
NITER = EPT // (CH * GRP * NSETS)
ROWS = NPAD // NS
WROWS = 640
WPAD = NS * WROWS
ZR = 40
ZN = 160
NSPS = 6
EPAD = NS * EPT


def _mlp_body(x_ref, w1_ref, b1_ref, w2_ref, b2_ref, o_ref):
    h = jnp.dot(x_ref[...], w1_ref[...], preferred_element_type=jnp.float32)
    h = jnp.maximum(h + b1_ref[...], 0.0)
    o_ref[...] = (
        jnp.dot(h, w2_ref[...], preferred_element_type=jnp.float32) + b2_ref[...]
    )


def _mlp(xpad, W1, b1, W2, b2):
    blocks = 8
    br = NPAD // blocks
    return pl.pallas_call(
        _mlp_body,
        grid=(blocks,),
        in_specs=[
            pl.BlockSpec((br, 128), lambda i: (i, 0)),
            pl.BlockSpec((128, 64), lambda i: (0, 0)),
            pl.BlockSpec((1, 64), lambda i: (0, 0)),
            pl.BlockSpec((64, F), lambda i: (0, 0)),
            pl.BlockSpec((1, F), lambda i: (0, 0)),
        ],
        out_specs=pl.BlockSpec((br, F), lambda i: (i, 0)),
        out_shape=jax.ShapeDtypeStruct((NPAD, F), jnp.float32),
    )(xpad, W1, b1.reshape(1, 64), W2, b2.reshape(1, F))


def _lsm_body(a_ref, o_ref):
    a = a_ref[...]
    m = jnp.max(a, axis=1, keepdims=True)
    e = jnp.exp(a - m)
    s = jnp.sum(e, axis=1, keepdims=True)
    o_ref[...] = a - m - jnp.log(s)


def _log_softmax(acc):
    blocks = 10
    br = N // blocks
    return pl.pallas_call(
        _lsm_body,
        grid=(blocks,),
        in_specs=[pl.BlockSpec((br, F), lambda i: (i, 0))],
        out_specs=pl.BlockSpec((br, F), lambda i: (i, 0)),
        out_shape=jax.ShapeDtypeStruct((N, F), jnp.float32),
    )(acc)


def _gpr_body(
    h_hbm, src_hbm, dst_hbm, tb_hbm, zn_hbm, zw_hbm, ones_hbm,
    acc_hbm, ubuf_hbm,
    w_sp, deg_sp, u_sp,
    sbuf, dbuf, grow, tbv, onesv, znv, zwv, dinvv, dinvb, wv, accv,
    gs0, gs1, gs2, gs3, ss0, ss1, ss2, ss3,
):
    s = lax.axis_index("s")
    base = s * ROWS
    wbase = s * WROWS

    pltpu.sync_copy(tb_hbm, tbv)
    pltpu.sync_copy(ones_hbm, onesv)
    pltpu.sync_copy(zn_hbm, znv)
    pltpu.sync_copy(zw_hbm, zwv)
    pltpu.sync_copy(src_hbm.at[s], sbuf)
    pltpu.sync_copy(dst_hbm.at[s], dbuf)
    for q in range(WROWS // ZN):
        pltpu.sync_copy(znv, deg_sp.at[pl.ds(wbase + q * ZN, ZN)])
    plsc.subcore_barrier()

    def deg_chunk(j, carry):
        pltpu.sync_copy(onesv, deg_sp.at[dbuf.at[j]], add=True)
        return carry
    lax.fori_loop(0, EPT // CH, deg_chunk, None)
    plsc.subcore_barrier()

    pltpu.sync_copy(deg_sp.at[pl.ds(base, ROWS)], dinvv)
    def newton(i, carry):
        x = dinvv[pl.ds(i * 16, 16)] + 1.0
        b = lax.bitcast_convert_type(x, jnp.int32)
        y = lax.bitcast_convert_type(
            jnp.int32(0x5F3759DF) - jnp.right_shift(b, 1), jnp.float32
        )
        for _ in range(3):
            y = y * (1.5 - 0.5 * x * y * y)
        dinvv[pl.ds(i * 16, 16)] = y
        return carry
    lax.fori_loop(0, ROWS // 16, newton, None)

    pltpu.sync_copy(h_hbm.at[pl.ds(base, ROWS)], wv)
    t0 = tbv[0]
    def initrow(r4, carry):
        for q in range(4):
            r = r4 * 4 + q
            dv = plsc.load_gather(dinvv, [jnp.full((16,), r, jnp.int32)])
            dinvb[r] = dv
            hrow = wv[r]
            wv[r] = dv * hrow
            accv[r] = t0 * hrow
        return carry
    lax.fori_loop(0, ROWS // 4, initrow, None)
    pltpu.sync_copy(wv, ubuf_hbm.at[pl.ds(base, ROWS)])
    pltpu.sync_copy(wv, u_sp.at[pl.ds(base, ROWS)])
    plsc.subcore_barrier()

    def round_body(k, carry):
        for q in range(WROWS // ZR):
            pltpu.async_copy(zwv, w_sp.at[pl.ds(wbase + q * ZR, ZR)], gs0)
        for q in range(WROWS // ZR):
            pltpu.make_async_copy(zwv, w_sp.at[pl.ds(wbase + q * ZR, ZR)], gs0).wait()
        plsc.subcore_barrier()

        gsems = (gs0, gs1, gs2, gs3)
        ssems = (ss0, ss1, ss2, ss3)

        def gfire(j, b):
            @pl.when(j < NSPS)
            def _():
                for i in range(GRP):
                    pltpu.async_copy(
                        u_sp.at[sbuf.at[j * GRP + i]], grow.at[b, i], gsems[b]
                    )
            @pl.when(j >= NSPS)
            def _():
                for i in range(GRP):
                    pltpu.async_copy(
                        ubuf_hbm.at[sbuf.at[j * GRP + i]], grow.at[b, i], gsems[b]
                    )

        def gdrain(b):
            for i in range(GRP):
                pltpu.make_async_copy(
                    ubuf_hbm.at[pl.ds(0, CH)], grow.at[b, i], gsems[b]
                ).wait()

        def sfire(j, b):
            for i in range(GRP):
                pltpu.async_copy(
                    grow.at[b, i], w_sp.at[dbuf.at[j * GRP + i]], ssems[b],
                    add=True,
                )

        def sdrain(b):
            for i in range(GRP):
                pltpu.make_async_copy(
                    grow.at[b, i], w_sp.at[pl.ds(0, CH)], ssems[b]
                ).wait()

        for b in range(NSETS):
            gfire(b, b)

        def rot(it, c):
            j0 = it * NSETS
            for b in range(NSETS):
                gdrain(b)
                sfire(j0 + b, b)
            for b in range(NSETS):
                sdrain(b)
                @pl.when(it < NITER - 1)
                def _():
                    gfire(j0 + NSETS + b, b)
            return c
        lax.fori_loop(0, NITER, rot, None)
        plsc.subcore_barrier()

        pltpu.sync_copy(w_sp.at[pl.ds(base, ROWS)], wv)
        pltpu.sync_copy(u_sp.at[pl.ds(base, CH)], grow.at[0, 0])
        pltpu.sync_copy(u_sp.at[pl.ds(base + CH, CH)], grow.at[0, 1])
        pltpu.sync_copy(
            u_sp.at[pl.ds(base + 2 * CH, ROWS - 2 * CH)],
            grow.at[1, 0].at[pl.ds(0, ROWS - 2 * CH)],
        )
        tk = tbv[k + 1]
        def mkseg(seg, off, nrows):
            def noderow(r4, c):
                for q in range(4):
                    rl = r4 * 4 + q
                    r = off + rl
                    z = dinvb[r] * (wv[r] + seg[rl])
                    accv[r] = accv[r] + tk * z
                    wv[r] = dinvb[r] * z
                return c
            lax.fori_loop(0, nrows // 4, noderow, None)
        mkseg(grow.at[0, 0], 0, CH)
        mkseg(grow.at[0, 1], CH, CH)
        mkseg(grow.at[1, 0], 2 * CH, ROWS - 2 * CH)
        pltpu.sync_copy(wv, ubuf_hbm.at[pl.ds(base, ROWS)])
        pltpu.sync_copy(wv, u_sp.at[pl.ds(base, ROWS)])
        plsc.subcore_barrier()
        return carry
    lax.fori_loop(0, K, round_body, None)

    pltpu.sync_copy(accv, acc_hbm.at[pl.ds(base, ROWS)])


def _gpr_sc(h, src3, dst3, tb, zn, zw, ones):
    mesh = plsc.VectorSubcoreMesh(
        core_axis_name="c", subcore_axis_name="s", num_cores=1
    )
    f = pl.kernel(
        _gpr_body,
        compiler_params=pltpu.CompilerParams(
            needs_layout_passes=False, use_tc_tiling_on_sc=False
        ),
        out_type=(
            jax.ShapeDtypeStruct((NPAD, F), jnp.float32),
            jax.ShapeDtypeStruct((NPAD, F), jnp.float32),
        ),
        mesh=mesh,
        scratch_types=[
            pltpu.VMEM_SHARED((WPAD, F), jnp.float32),
            pltpu.VMEM_SHARED((WPAD,), jnp.float32),
            pltpu.VMEM_SHARED((WPAD, F), jnp.float32),
            pltpu.VMEM((EPT // CH, CH), jnp.int32),
            pltpu.VMEM((EPT // CH, CH), jnp.int32),
            pltpu.VMEM((NSETS, GRP, CH, F), jnp.float32),
            pltpu.VMEM((16, F), jnp.float32),
            pltpu.VMEM((CH,), jnp.float32),
            pltpu.VMEM((ZN,), jnp.float32),
            pltpu.VMEM((ZR, F), jnp.float32),
            pltpu.VMEM((ROWS,), jnp.float32),
            pltpu.VMEM((ROWS, F), jnp.float32),
            pltpu.VMEM((ROWS, F), jnp.float32),
            pltpu.VMEM((ROWS, F), jnp.float32),
        ] + [pltpu.SemaphoreType.DMA] * 8,
    )
    return f(h, src3, dst3, tb, zn, zw, ones)


def kernel(x, edge_index, W1, b1, W2, b2, temp):
    xpad = jnp.pad(x.astype(jnp.float32), ((0, NPAD - N), (0, 0)))
    h = _mlp(xpad, W1, b1, W2, b2)

    src = edge_index[0].astype(jnp.int32)
    dst = edge_index[1].astype(jnp.int32)
    npad_e = EPAD - E
    pidx = jnp.arange(npad_e, dtype=jnp.int32)
    src3 = jnp.concatenate([src, pidx % N]).reshape(NS, EPT // CH, CH)
    dst3 = jnp.concatenate([dst, N + pidx % (NPAD - N)]).reshape(NS, EPT // CH, CH)

    tb = jnp.zeros((16, F), jnp.float32)
    tb = tb.at[: K + 1].set(jnp.broadcast_to(temp.astype(jnp.float32)[:, None], (K + 1, F)))
    zn = jnp.zeros((ZN,), jnp.float32)
    zw = jnp.zeros((ZR, F), jnp.float32)
    ones = jnp.ones((CH,), jnp.float32)

    acc, _ = _gpr_sc(h, src3, dst3, tb, zn, zw, ones)
    return _log_softmax(acc)


# speedup vs baseline: 1.0246x; 1.0246x over previous
import functools

import jax
import jax.numpy as jnp
from jax import lax
from jax.experimental import pallas as pl
from jax.experimental.pallas import tpu as pltpu
from jax.experimental.pallas import tpu_sc as plsc

N = 10000
NPAD = 10240
E = 320000
F = 16
K = 10
NS = 16
EPT = 20480
CH = 256
GRP = 2
NSETS = 4
NITER = EPT // (CH * GRP * NSETS)
ROWS = NPAD // NS
WROWS = 640
WPAD = NS * WROWS
ZR = 160
EPAD = NS * EPT


def _mlp_body(x_ref, w1_ref, b1_ref, w2_ref, b2_ref, o_ref):
    h = jnp.dot(x_ref[...], w1_ref[...], preferred_element_type=jnp.float32)
    h = jnp.maximum(h + b1_ref[...], 0.0)
    o_ref[...] = (
        jnp.dot(h, w2_ref[...], preferred_element_type=jnp.float32) + b2_ref[...]
    )


def _mlp(xpad, W1, b1, W2, b2):
    blocks = 8
    br = NPAD // blocks
    return pl.pallas_call(
        _mlp_body,
        grid=(blocks,),
        in_specs=[
            pl.BlockSpec((br, 128), lambda i: (i, 0)),
            pl.BlockSpec((128, 64), lambda i: (0, 0)),
            pl.BlockSpec((1, 64), lambda i: (0, 0)),
            pl.BlockSpec((64, F), lambda i: (0, 0)),
            pl.BlockSpec((1, F), lambda i: (0, 0)),
        ],
        out_specs=pl.BlockSpec((br, F), lambda i: (i, 0)),
        out_shape=jax.ShapeDtypeStruct((NPAD, F), jnp.float32),
    )(xpad, W1, b1.reshape(1, 64), W2, b2.reshape(1, F))


def _lsm_body(a_ref, o_ref):
    a = a_ref[...]
    m = jnp.max(a, axis=1, keepdims=True)
    e = jnp.exp(a - m)
    s = jnp.sum(e, axis=1, keepdims=True)
    o_ref[...] = a - m - jnp.log(s)


def _log_softmax(acc):
    blocks = 10
    br = N // blocks
    return pl.pallas_call(
        _lsm_body,
        grid=(blocks,),
        in_specs=[pl.BlockSpec((br, F), lambda i: (i, 0))],
        out_specs=pl.BlockSpec((br, F), lambda i: (i, 0)),
        out_shape=jax.ShapeDtypeStruct((N, F), jnp.float32),
    )(acc)


def _gpr_body(
    h_hbm, src_hbm, dst_hbm, tb_hbm, zn_hbm, zw_hbm, ones_hbm,
    acc_hbm, ubuf_hbm,
    w_sp, deg_sp,
    sbuf, dbuf, grow, tbv, onesv, znv, zwv, dinvv, dinvb, wv, uv, accv,
    gs0, gs1, gs2, gs3, ss0, ss1, ss2, ss3,
):
    s = lax.axis_index("s")
    base = s * ROWS
    wbase = s * WROWS

    pltpu.sync_copy(tb_hbm, tbv)
    pltpu.sync_copy(ones_hbm, onesv)
    pltpu.sync_copy(zn_hbm, znv)
    pltpu.sync_copy(zw_hbm, zwv)
    pltpu.sync_copy(src_hbm.at[s], sbuf)
    pltpu.sync_copy(dst_hbm.at[s], dbuf)
    pltpu.sync_copy(znv, deg_sp.at[pl.ds(wbase, WROWS)])
    plsc.subcore_barrier()

    def deg_grp(g, carry):
        for i in range(8):
            pltpu.async_copy(onesv, deg_sp.at[dbuf.at[g * 8 + i]], gs0, add=True)
        for i in range(8):
            pltpu.make_async_copy(onesv, deg_sp.at[pl.ds(0, CH)], gs0).wait()
        return carry
    lax.fori_loop(0, EPT // CH // 8, deg_grp, None)
    plsc.subcore_barrier()

    pltpu.sync_copy(deg_sp.at[pl.ds(base, ROWS)], dinvv)
    def newton(i, carry):
        x = dinvv[pl.ds(i * 16, 16)] + 1.0
        b = lax.bitcast_convert_type(x, jnp.int32)
        y = lax.bitcast_convert_type(
            jnp.int32(0x5F3759DF) - jnp.right_shift(b, 1), jnp.float32
        )
        for _ in range(3):
            y = y * (1.5 - 0.5 * x * y * y)
        dinvv[pl.ds(i * 16, 16)] = y
        return carry
    lax.fori_loop(0, ROWS // 16, newton, None)

    pltpu.sync_copy(h_hbm.at[pl.ds(base, ROWS)], wv)
    t0 = tbv[0]
    def initrow(r4, carry):
        for q in range(4):
            r = r4 * 4 + q
            dv = plsc.load_gather(dinvv, [jnp.full((16,), r, jnp.int32)])
            dinvb[r] = dv
            hrow = wv[r]
            uv[r] = dv * hrow
            accv[r] = t0 * hrow
        return carry
    lax.fori_loop(0, ROWS // 4, initrow, None)
    pltpu.sync_copy(uv, ubuf_hbm.at[pl.ds(base, ROWS)])
    plsc.subcore_barrier()

    def round_body(k, carry):
        for q in range(WROWS // ZR):
            pltpu.async_copy(zwv, w_sp.at[pl.ds(wbase + q * ZR, ZR)], gs0)
        for q in range(WROWS // ZR):
            pltpu.make_async_copy(zwv, w_sp.at[pl.ds(wbase + q * ZR, ZR)], gs0).wait()
        plsc.subcore_barrier()

        gsems = (gs0, gs1, gs2, gs3)
        ssems = (ss0, ss1, ss2, ss3)

        def gfire(j, b):
            for i in range(GRP):
                pltpu.async_copy(
                    ubuf_hbm.at[sbuf.at[j * GRP + i]], grow.at[b, i], gsems[b]
                )

        def gdrain(b):
            for i in range(GRP):
                pltpu.make_async_copy(
                    ubuf_hbm.at[pl.ds(0, CH)], grow.at[b, i], gsems[b]
                ).wait()

        def sfire(j, b):
            for i in range(GRP):
                pltpu.async_copy(
                    grow.at[b, i], w_sp.at[dbuf.at[j * GRP + i]], ssems[b],
                    add=True,
                )

        def sdrain(b):
            for i in range(GRP):
                pltpu.make_async_copy(
                    grow.at[b, i], w_sp.at[pl.ds(0, CH)], ssems[b]
                ).wait()

        for b in range(NSETS):
            gfire(b, b)

        def rot(it, c):
            j0 = it * NSETS
            for b in range(NSETS):
                gdrain(b)
                sfire(j0 + b, b)
            for b in range(NSETS):
                sdrain(b)
                @pl.when(it < NITER - 1)
                def _():
                    gfire(j0 + NSETS + b, b)
            return c
        lax.fori_loop(0, NITER, rot, None)
        plsc.subcore_barrier()

        pltpu.sync_copy(w_sp.at[pl.ds(base, ROWS)], wv)
        tk = tbv[k + 1]
        def noderow(r4, c):
            for q in range(4):
                r = r4 * 4 + q
                z = dinvb[r] * (wv[r] + uv[r])
                accv[r] = accv[r] + tk * z
                uv[r] = dinvb[r] * z
            return c
        lax.fori_loop(0, ROWS // 4, noderow, None)
        pltpu.sync_copy(uv, ubuf_hbm.at[pl.ds(base, ROWS)])
        plsc.subcore_barrier()
        return carry
    lax.fori_loop(0, K, round_body, None)

    pltpu.sync_copy(accv, acc_hbm.at[pl.ds(base, ROWS)])


def _gpr_sc(h, src3, dst3, tb, zn, zw, ones):
    mesh = plsc.VectorSubcoreMesh(
        core_axis_name="c", subcore_axis_name="s", num_cores=1
    )
    f = pl.kernel(
        _gpr_body,
        compiler_params=pltpu.CompilerParams(
            needs_layout_passes=False, use_tc_tiling_on_sc=False
        ),
        out_type=(
            jax.ShapeDtypeStruct((NPAD, F), jnp.float32),
            jax.ShapeDtypeStruct((NPAD, F), jnp.float32),
        ),
        mesh=mesh,
        scratch_types=[
            pltpu.VMEM_SHARED((WPAD, F), jnp.float32),
            pltpu.VMEM_SHARED((WPAD,), jnp.float32),
            pltpu.VMEM((EPT // CH, CH), jnp.int32),
            pltpu.VMEM((EPT // CH, CH), jnp.int32),
            pltpu.VMEM((NSETS, GRP, CH, F), jnp.float32),
            pltpu.VMEM((16, F), jnp.float32),
            pltpu.VMEM((CH,), jnp.float32),
            pltpu.VMEM((WROWS,), jnp.float32),
            pltpu.VMEM((ZR, F), jnp.float32),
            pltpu.VMEM((ROWS,), jnp.float32),
            pltpu.VMEM((ROWS, F), jnp.float32),
            pltpu.VMEM((ROWS, F), jnp.float32),
            pltpu.VMEM((ROWS, F), jnp.float32),
            pltpu.VMEM((ROWS, F), jnp.float32),
        ] + [pltpu.SemaphoreType.DMA] * 8,
    )
    return f(h, src3, dst3, tb, zn, zw, ones)


def kernel(x, edge_index, W1, b1, W2, b2, temp):
    xpad = jnp.pad(x.astype(jnp.float32), ((0, NPAD - N), (0, 0)))
    h = _mlp(xpad, W1, b1, W2, b2)

    src = edge_index[0].astype(jnp.int32)
    dst = edge_index[1].astype(jnp.int32)
    npad_e = EPAD - E
    pidx = jnp.arange(npad_e, dtype=jnp.int32)
    src3 = jnp.concatenate([src, pidx % N]).reshape(NS, EPT // CH, CH)
    dst3 = jnp.concatenate([dst, N + pidx % (NPAD - N)]).reshape(NS, EPT // CH, CH)

    tb = jnp.zeros((16, F), jnp.float32)
    tb = tb.at[: K + 1].set(jnp.broadcast_to(temp.astype(jnp.float32)[:, None], (K + 1, F)))
    zn = jnp.zeros((WROWS,), jnp.float32)
    zw = jnp.zeros((ZR, F), jnp.float32)
    ones = jnp.ones((CH,), jnp.float32)

    acc, _ = _gpr_sc(h, src3, dst3, tb, zn, zw, ones)
    return _log_softmax(acc)
